# Initial kernel scaffold; baseline (speedup 1.0000x reference)
import jax
import jax.numpy as jnp
from jax.experimental import pallas as pl


def kernel(x, edge_index, batch, cW1, cb1, cW2, cb2, cW3, cb3, mW1, mb1, mW2, mb2, mW3, mb3):
    raise NotImplementedError("write your pallas kernel here")


# speedup vs baseline: 5.8434x; 5.8434x over previous
import functools
import jax
import jax.numpy as jnp
from jax import lax
from jax.experimental import pallas as pl
from jax.experimental.pallas import tpu as pltpu
from jax.experimental.pallas import tpu_sc as plsc

N = 10000
NG = 64
NPAD = 10240
RPT = NPAD // 16
TRASH = N
EC = 128
NCH2 = 79
E_PAD = 32 * NCH2 * EC
NCH1 = E_PAD // (16 * EC)
NBLK = 10

f32 = jnp.float32
i32 = jnp.int32

_MESH1 = plsc.VectorSubcoreMesh(core_axis_name="c", subcore_axis_name="s",
                                num_cores=1)
_MESH2 = plsc.VectorSubcoreMesh(core_axis_name="c", subcore_axis_name="s")


def _prep_body(row_hbm, col_hbm, colp_hbm, dis_hbm,
               rbuf, cbuf, cpbuf, deg_v, red_v, acc_v, stage_sh):
    s = lax.axis_index("s")
    base = s * (NCH1 * EC)
    zero16 = jnp.zeros((16,), f32)

    def zdeg(i, _):
        deg_v[pl.ds(i * 16, 16)] = zero16
        return 0
    lax.fori_loop(0, NPAD // 16, zdeg, 0)

    def chunk(ch, _):
        off = base + ch * EC
        pltpu.sync_copy(row_hbm.at[pl.ds(off, EC)], rbuf)
        pltpu.sync_copy(col_hbm.at[pl.ds(off, EC)], cbuf)
        for j in range(EC // 16):
            r = rbuf[pl.ds(j * 16, 16)]
            cc = cbuf[pl.ds(j * 16, 16)]
            m = r == cc
            cpbuf[pl.ds(j * 16, 16)] = jnp.where(m, TRASH, cc).astype(i32)
            plsc.addupdate_scatter(deg_v, [r],
                                   jnp.where(m, 0.0, 1.0).astype(f32))
        pltpu.sync_copy(cpbuf, colp_hbm.at[pl.ds(off, EC)])
        return 0
    lax.fori_loop(0, NCH1, chunk, 0)

    pltpu.sync_copy(deg_v, stage_sh.at[s])
    plsc.subcore_barrier()
    rb = s * RPT
    pltpu.sync_copy(stage_sh.at[0, pl.ds(rb, RPT)], acc_v)
    for k in range(1, 16):
        pltpu.sync_copy(stage_sh.at[k, pl.ds(rb, RPT)], red_v)

        def addk(i, _):
            acc_v[pl.ds(i * 16, 16)] = (acc_v[pl.ds(i * 16, 16)]
                                        + red_v[pl.ds(i * 16, 16)])
            return 0
        lax.fori_loop(0, RPT // 16, addk, 0)

    def newt(i, _):
        d = acc_v[pl.ds(i * 16, 16)]
        bits = jnp.int32(0x5F3759DF) - (plsc.bitcast(d, i32) >> 1)
        y = plsc.bitcast(bits, f32)
        for _ in range(4):
            y = y * (1.5 - 0.5 * d * y * y)
        acc_v[pl.ds(i * 16, 16)] = jnp.where(d > 0, y, 0.0).astype(f32)
        return 0
    lax.fori_loop(0, RPT // 16, newt, 0)
    pltpu.sync_copy(acc_v, dis_hbm.at[pl.ds(rb, RPT)])


_prep = pl.kernel(
    _prep_body,
    out_type=(jax.ShapeDtypeStruct((E_PAD,), i32),
              jax.ShapeDtypeStruct((NPAD,), f32)),
    mesh=_MESH1,
    compiler_params=pltpu.CompilerParams(needs_layout_passes=False),
    scratch_types=[
        pltpu.VMEM((EC,), i32),
        pltpu.VMEM((EC,), i32),
        pltpu.VMEM((EC,), i32),
        pltpu.VMEM((NPAD,), f32),
        pltpu.VMEM((RPT,), f32),
        pltpu.VMEM((RPT,), f32),
        pltpu.VMEM_SHARED((16, NPAD), f32),
    ],
)


@functools.lru_cache(maxsize=None)
def _make_spmm(F):
    def body(xs_hbm, row_hbm, colp_hbm, zeros_hbm, ypart_hbm,
             idx_r, idx_c, gbuf, out_sh, sem):
        c = lax.axis_index("c")
        s = lax.axis_index("s")
        wid = c * 16 + s
        ebase = wid * (NCH2 * EC)
        rb = s * RPT
        pltpu.sync_copy(zeros_hbm.at[pl.ds(rb, RPT)],
                        out_sh.at[pl.ds(rb, RPT)])
        plsc.subcore_barrier()

        def chunk(ch, _):
            off = ebase + ch * EC
            pltpu.sync_copy(row_hbm.at[pl.ds(off, EC)], idx_r)
            pltpu.sync_copy(colp_hbm.at[pl.ds(off, EC)], idx_c)
            pltpu.async_copy(xs_hbm.at[idx_r], gbuf, sem).wait()
            pltpu.sync_copy(gbuf, out_sh.at[idx_c], add=True)
            return 0
        lax.fori_loop(0, NCH2, chunk, 0)

        plsc.subcore_barrier()
        pltpu.sync_copy(out_sh.at[pl.ds(rb, RPT)],
                        ypart_hbm.at[pl.ds(c * NPAD + rb, RPT)])

    return pl.kernel(
        body,
        out_type=jax.ShapeDtypeStruct((2 * NPAD, F), f32),
        mesh=_MESH2,
        compiler_params=pltpu.CompilerParams(needs_layout_passes=False,
                                             use_tc_tiling_on_sc=False),
        scratch_types=[
            pltpu.VMEM((EC,), i32),
            pltpu.VMEM((EC,), i32),
            pltpu.VMEM((EC, F), f32),
            pltpu.VMEM_SHARED((NPAD, F), f32),
            pltpu.SemaphoreType.DMA,
        ],
    )


def _scale_body(x_ref, d_ref, o_ref):
    o_ref[...] = x_ref[...] * d_ref[...]


def _scale(x, dis2):
    F = x.shape[1]
    return pl.pallas_call(
        _scale_body,
        grid=(NBLK,),
        in_specs=[pl.BlockSpec((N // NBLK, F), lambda i: (i, 0)),
                  pl.BlockSpec((N // NBLK, 1), lambda i: (i, 0))],
        out_specs=pl.BlockSpec((N // NBLK, F), lambda i: (i, 0)),
        out_shape=jax.ShapeDtypeStruct((N, F), f32),
    )(x, dis2)


def _mid_body(y0_ref, y1_ref, d_ref, t1_ref, xs_ref):
    d = d_ref[...]
    t1 = -d * (y0_ref[0] + y1_ref[0])
    t1_ref[...] = t1
    xs_ref[...] = d * t1


def _mid(ypart, dis2):
    F = ypart.shape[2]
    R = N // NBLK
    return pl.pallas_call(
        _mid_body,
        grid=(NBLK,),
        in_specs=[pl.BlockSpec((1, R, F), lambda i: (0, i, 0)),
                  pl.BlockSpec((1, R, F), lambda i: (1, i, 0)),
                  pl.BlockSpec((R, 1), lambda i: (i, 0))],
        out_specs=[pl.BlockSpec((R, F), lambda i: (i, 0)),
                   pl.BlockSpec((R, F), lambda i: (i, 0))],
        out_shape=(jax.ShapeDtypeStruct((N, F), f32),
                   jax.ShapeDtypeStruct((N, F), f32)),
    )(ypart, ypart, dis2)


def _combine(ypart, dis2, t0, t1, cw, cb2, act, with_xs):
    Fi, Fo = cw.shape[1], cw.shape[2]
    R = N // NBLK

    def body(*refs):
        if with_xs:
            (y0_ref, y1_ref, d_ref, t0_ref, t1_ref,
             w0, w1, w2, b_ref, o_ref, xs_ref) = refs
        else:
            (y0_ref, y1_ref, d_ref, t0_ref, t1_ref,
             w0, w1, w2, b_ref, o_ref) = refs
        d = d_ref[...]
        t0v = t0_ref[...]
        t2 = -2.0 * d * (y0_ref[0] + y1_ref[0]) - t0v
        o = (jnp.dot(t0v, w0[0], preferred_element_type=f32)
             + jnp.dot(t1_ref[...], w1[0], preferred_element_type=f32)
             + jnp.dot(t2, w2[0], preferred_element_type=f32)
             + b_ref[...])
        if act:
            o = jnp.where(o > 0, o, 0.01 * o)
        o_ref[...] = o
        if with_xs:
            xs_ref[...] = d * o

    out_shape = [jax.ShapeDtypeStruct((N, Fo), f32)]
    out_specs = [pl.BlockSpec((R, Fo), lambda i: (i, 0))]
    if with_xs:
        out_shape.append(jax.ShapeDtypeStruct((N, Fo), f32))
        out_specs.append(pl.BlockSpec((R, Fo), lambda i: (i, 0)))
    res = pl.pallas_call(
        body,
        grid=(NBLK,),
        in_specs=[pl.BlockSpec((1, R, Fi), lambda i: (0, i, 0)),
                  pl.BlockSpec((1, R, Fi), lambda i: (1, i, 0)),
                  pl.BlockSpec((R, 1), lambda i: (i, 0)),
                  pl.BlockSpec((R, Fi), lambda i: (i, 0)),
                  pl.BlockSpec((R, Fi), lambda i: (i, 0)),
                  pl.BlockSpec((1, Fi, Fo), lambda i: (0, 0, 0)),
                  pl.BlockSpec((1, Fi, Fo), lambda i: (1, 0, 0)),
                  pl.BlockSpec((1, Fi, Fo), lambda i: (2, 0, 0)),
                  pl.BlockSpec((1, Fo), lambda i: (0, 0))],
        out_specs=out_specs,
        out_shape=out_shape,
    )(ypart, ypart, dis2, t0, t1, cw, cw, cw, cb2)
    return res if with_xs else res[0]


def _pool_body(h_ref, b_ref, o_ref, gacc):
    i = pl.program_id(0)

    @pl.when(i == 0)
    def _():
        gacc[...] = jnp.full((NG, 256), -jnp.inf, f32)

    R = N // NBLK
    bmin = b_ref[0, 0]
    bmax = b_ref[R - 1, 0]
    hb = h_ref[...]
    bb = b_ref[...]

    def gbody(g, _):
        v = jnp.where(bb == g, hb, -jnp.inf)
        red = jnp.max(v, axis=0, keepdims=True)
        gacc[pl.ds(g, 1), :] = jnp.maximum(gacc[pl.ds(g, 1), :], red)
        return 0
    lax.fori_loop(bmin, bmax + 1, gbody, 0)

    @pl.when(i == NBLK - 1)
    def _():
        o_ref[...] = jnp.where(jnp.isfinite(gacc[...]), gacc[...], 0.0)


def _pool(h, batch2):
    R = N // NBLK
    return pl.pallas_call(
        _pool_body,
        grid=(NBLK,),
        in_specs=[pl.BlockSpec((R, 256), lambda i: (i, 0)),
                  pl.BlockSpec((R, 1), lambda i: (i, 0))],
        out_specs=pl.BlockSpec((NG, 256), lambda i: (0, 0)),
        out_shape=jax.ShapeDtypeStruct((NG, 256), f32),
        scratch_shapes=[pltpu.VMEM((NG, 256), f32)],
    )(h, batch2)


def _mlp_body(g_ref, w1, b1, w2, b2, w3, b3, o_ref):
    z = jnp.dot(g_ref[...], w1[...], preferred_element_type=f32) + b1[...]
    z = jnp.maximum(z, 0.0)
    z = jnp.dot(z, w2[...], preferred_element_type=f32) + b2[...]
    z = jnp.maximum(z, 0.0)
    o_ref[...] = jnp.dot(z, w3[...], preferred_element_type=f32) + b3[...]


def _mlp(g, mW1, mb1, mW2, mb2, mW3, mb3):
    return pl.pallas_call(
        _mlp_body,
        out_shape=jax.ShapeDtypeStruct((NG, 4), f32),
    )(g, mW1, mb1.reshape(1, -1), mW2, mb2.reshape(1, -1),
      mW3, mb3.reshape(1, -1))


def kernel(x, edge_index, batch, cW1, cb1, cW2, cb2, cW3, cb3,
           mW1, mb1, mW2, mb2, mW3, mb3):
    E = edge_index.shape[1]
    pad = jnp.zeros((E_PAD - E,), i32)
    rowp = jnp.concatenate([edge_index[0].astype(i32), pad])
    colq = jnp.concatenate([edge_index[1].astype(i32), pad])

    colp, dis = _prep(rowp, colq)
    dis2 = dis[:N].reshape(N, 1)

    z128 = jnp.zeros((NPAD, 128), f32)
    z64 = jnp.zeros((NPAD, 64), f32)
    zeros_by_f = {128: z128, 64: z64}

    t0 = x
    xs = _scale(x, dis2)
    for cw, cb, last in ((cW1, cb1, False), (cW2, cb2, False), (cW3, cb3, True)):
        F = cw.shape[1]
        spmm = _make_spmm(F)
        zf = zeros_by_f[F]
        y1 = spmm(xs, rowp, colp, zf).reshape(2, NPAD, F)
        t1, xs1 = _mid(y1, dis2)
        y2 = spmm(xs1, rowp, colp, zf).reshape(2, NPAD, F)
        if last:
            h = _combine(y2, dis2, t0, t1, cw, cb.reshape(1, -1),
                         act=False, with_xs=False)
        else:
            t0, xs = _combine(y2, dis2, t0, t1, cw, cb.reshape(1, -1),
                              act=True, with_xs=True)

    g = _pool(h, batch.reshape(N, 1).astype(i32))
    return _mlp(g, mW1, mb1, mW2, mb2, mW3, mb3)
